# baseline (speedup 1.0000x reference)
import jax
import jax.numpy as jnp
from jax.experimental import pallas as pl
from jax.experimental.pallas import tpu as pltpu

_N = 4


def _copy_kernel(x_ref, o_ref, *rest):
    bufs = rest[:_N]
    sin = rest[_N:2 * _N]
    sout = rest[2 * _N:]
    h = x_ref.shape[0] // _N
    ins = [
        pltpu.make_async_copy(x_ref.at[pl.ds(i * h, h)], bufs[i], sin[i])
        for i in range(_N)
    ]
    outs = [
        pltpu.make_async_copy(bufs[i], o_ref.at[pl.ds(i * h, h)], sout[i])
        for i in range(_N)
    ]
    ins[0].start()
    for i in range(_N):
        if i + 1 < _N:
            ins[i + 1].start()
        ins[i].wait()
        outs[i].start()
    for i in range(_N):
        outs[i].wait()


def kernel(input):
    t, b, s = input.shape
    return pl.pallas_call(
        _copy_kernel,
        out_shape=jax.ShapeDtypeStruct(input.shape, input.dtype),
        in_specs=[pl.BlockSpec(memory_space=pl.ANY)],
        out_specs=pl.BlockSpec(memory_space=pl.ANY),
        scratch_shapes=(
            [pltpu.VMEM((t // _N, b, s), input.dtype) for _ in range(_N)]
            + [pltpu.SemaphoreType.DMA] * (2 * _N)
        ),
    )(input)


# speedup vs baseline: 1.4668x; 1.4668x over previous
import jax
import jax.numpy as jnp
from jax.experimental import pallas as pl
from jax.experimental.pallas import tpu as pltpu


def _copy_kernel(x_ref, o_ref, buf0, buf1, si0, si1, so0, so1):
    h = x_ref.shape[0] // 2
    in0 = pltpu.make_async_copy(x_ref.at[pl.ds(0, h)], buf0, si0)
    in1 = pltpu.make_async_copy(x_ref.at[pl.ds(h, h)], buf1, si1)
    in0.start()
    in1.start()
    in0.wait()
    out0 = pltpu.make_async_copy(buf0, o_ref.at[pl.ds(0, h)], so0)
    out0.start()
    in1.wait()
    out1 = pltpu.make_async_copy(buf1, o_ref.at[pl.ds(h, h)], so1)
    out1.start()
    out0.wait()
    out1.wait()


def kernel(input):
    t, b, s = input.shape
    return pl.pallas_call(
        _copy_kernel,
        out_shape=jax.ShapeDtypeStruct(input.shape, input.dtype),
        in_specs=[pl.BlockSpec(memory_space=pl.ANY)],
        out_specs=pl.BlockSpec(memory_space=pl.ANY),
        scratch_shapes=[
            pltpu.VMEM((t // 2, b, s), input.dtype),
            pltpu.VMEM((t // 2, b, s), input.dtype),
            pltpu.SemaphoreType.DMA,
            pltpu.SemaphoreType.DMA,
            pltpu.SemaphoreType.DMA,
            pltpu.SemaphoreType.DMA,
        ],
    )(input)
